# Initial kernel scaffold; baseline (speedup 1.0000x reference)
import jax
import jax.numpy as jnp
from jax.experimental import pallas as pl


def kernel(x_lc, batch_lc, W_e1, b_e1, W_e2, b_e2, W_c1, b_c1, g_c1, be_c1, W_c2, b_c2, g_c2, be_c2, W_c3, b_c3, g_c3, be_c3, W_o1, b_o1, W_o2, b_o2, W_o3, b_o3):
    raise NotImplementedError("write your pallas kernel here")


# speedup vs baseline: 9.5738x; 9.5738x over previous
import functools

import jax
import jax.numpy as jnp
from jax import lax
from jax.experimental import pallas as pl
from jax.experimental.pallas import tpu as pltpu
from jax.experimental.pallas import tpu_sc as plsc

_N = 10000
_NP = 10240
_K = 4
_H = 128
_RT = 512
_CT = 512
_R = _NP // _RT
_NJ = _NP // _CT
_NKROWS = float(_N * _K)

_SC_CORES = 2
_SC_SUBCORES = 16
_NW = _SC_CORES * _SC_SUBCORES
_E = _NP * _K
_EPW = _E // _NW
_GCH = 320
_NCH = _EPW // _GCH


def _elu(z):
    return jnp.where(z > 0, z, jnp.exp(jnp.minimum(z, 0.0)) - 1.0)


def _bf(v):
    return v.astype(jnp.bfloat16)


def _dotbf(x, w):
    return lax.dot_general(_bf(x), _bf(w), (((1,), (0,)), ((), ())),
                           preferred_element_type=jnp.float32)


def _enc_body(xlc_ref, we1_ref, be1_ref, we2_ref, be2_ref, wt_ref,
              bc_ref, x_ref, a_ref, sq_ref):
    h = _elu(_dotbf(xlc_ref[...], we1_ref[...]) + be1_ref[...])
    x = _elu(_dotbf(h, we2_ref[...]) + be2_ref[...])
    x_ref[...] = x
    a_ref[...] = _dotbf(x, wt_ref[...]) + bc_ref[...]
    sq_ref[...] = jnp.sum(x * x, axis=1, keepdims=True)


def _enc_call(xp, we1, be1, we2, be2, wt, bc):
    s = jax.ShapeDtypeStruct((_NP, _H), jnp.float32)
    return pl.pallas_call(
        _enc_body,
        grid=(_R,),
        in_specs=[
            pl.BlockSpec((_RT, 16), lambda r: (r, 0)),
            pl.BlockSpec((16, _H), lambda r: (0, 0)),
            pl.BlockSpec((1, _H), lambda r: (0, 0)),
            pl.BlockSpec((_H, _H), lambda r: (0, 0)),
            pl.BlockSpec((1, _H), lambda r: (0, 0)),
            pl.BlockSpec((_H, _H), lambda r: (0, 0)),
            pl.BlockSpec((1, _H), lambda r: (0, 0)),
        ],
        out_specs=[pl.BlockSpec((_RT, _H), lambda r: (r, 0))] * 2
        + [pl.BlockSpec((_RT, 1), lambda r: (r, 0))],
        out_shape=[s, s, jax.ShapeDtypeStruct((_NP, 1), jnp.float32)],
    )(xp, we1, be1, we2, be2, wt, bc)


def _knn_body(bounds_ref, xr_ref, xc_ref, sqr_ref, sqc_ref, rs_ref, re_ref,
              idx_ref, vals_scr, ids_scr):
    r = pl.program_id(0)
    j = pl.program_id(1)
    lo = bounds_ref[0, r]
    hi = bounds_ref[1, r]

    @pl.when(j == 0)
    def _init():
        vals_scr[...] = jnp.full((_RT, _K), jnp.inf, jnp.float32)
        ids_scr[...] = jnp.zeros((_RT, _K), jnp.int32)

    @pl.when(lo + j <= hi)
    def _work():
        cb = jnp.minimum(lo + j, hi)
        rs = rs_ref[0]
        re = re_ref[0]
        sqr = sqr_ref[0]
        sqc = sqc_ref[0]
        d = sqr + sqc - 2.0 * lax.dot_general(
            _bf(xr_ref[...]), _bf(xc_ref[...]), (((1,), (1,)), ((), ())),
            preferred_element_type=jnp.float32)
        col0 = cb * _CT
        pos = lax.broadcasted_iota(jnp.int32, (_RT, _CT), 1)
        cols = col0 + pos
        d = jnp.where((cols < rs) | (cols >= re), jnp.inf, d)
        tv, ti = [], []
        for _ in range(_K):
            mn = jnp.min(d, axis=1, keepdims=True)
            am = jnp.min(jnp.where(d == mn, pos, _CT), axis=1, keepdims=True)
            tv.append(mn)
            ti.append(col0 + am)
            d = jnp.where(pos == am, jnp.inf, d)
        av = jnp.concatenate([vals_scr[...]] + tv, axis=1)
        ai = jnp.concatenate([ids_scr[...]] + ti, axis=1)
        pos8 = lax.broadcasted_iota(jnp.int32, (_RT, 2 * _K), 1)
        nv, ni = [], []
        for _ in range(_K):
            mn = jnp.min(av, axis=1, keepdims=True)
            am = jnp.min(jnp.where(av == mn, pos8, 2 * _K), axis=1,
                         keepdims=True)
            sel = pos8 == am
            nv.append(mn)
            ni.append(jnp.sum(jnp.where(sel, ai, 0), axis=1, keepdims=True))
            av = jnp.where(sel, jnp.inf, av)
        vals_scr[...] = jnp.concatenate(nv, axis=1)
        ids_scr[...] = jnp.concatenate(ni, axis=1)

    @pl.when(j == _NJ - 1)
    def _fin():
        v = vals_scr[...]
        ii = ids_scr[...]
        rs = rs_ref[0]
        re = re_ref[0]
        finite = v < jnp.float32(1e30)
        nval = jnp.sum(finite.astype(jnp.int32), axis=1, keepdims=True)
        slot = lax.broadcasted_iota(jnp.int32, (_RT, _K), 1)
        m = slot - nval
        fb = jnp.where(m < rs, m, re + (m - rs))
        fb = jnp.clip(fb, 0, _N - 1)
        idx_ref[...] = jnp.where(finite, ii, fb)


def _knn_call(bounds, x, sqa, sqb, rs3, re3):
    grid_spec = pltpu.PrefetchScalarGridSpec(
        num_scalar_prefetch=1,
        grid=(_R, _NJ),
        in_specs=[
            pl.BlockSpec((_RT, _H), lambda r, j, b: (r, 0)),
            pl.BlockSpec((_CT, _H),
                         lambda r, j, b: (jnp.minimum(b[0, r] + j, b[1, r]),
                                          0)),
            pl.BlockSpec((1, _RT, 1), lambda r, j, b: (r, 0, 0)),
            pl.BlockSpec((1, 1, _CT),
                         lambda r, j, b: (jnp.minimum(b[0, r] + j, b[1, r]),
                                          0, 0)),
            pl.BlockSpec((1, _RT, 1), lambda r, j, b: (r, 0, 0)),
            pl.BlockSpec((1, _RT, 1), lambda r, j, b: (r, 0, 0)),
        ],
        out_specs=pl.BlockSpec((_RT, _K), lambda r, j, b: (r, 0)),
        scratch_shapes=[pltpu.VMEM((_RT, _K), jnp.float32),
                        pltpu.VMEM((_RT, _K), jnp.int32)],
    )
    return pl.pallas_call(
        _knn_body,
        grid_spec=grid_spec,
        out_shape=jax.ShapeDtypeStruct((_NP, _K), jnp.int32),
    )(bounds, x, x, sqa, sqb, rs3, re3)


def _gather_call(x, idx_flat):
    mesh = plsc.VectorSubcoreMesh(core_axis_name="c", subcore_axis_name="s")

    @functools.partial(
        pl.kernel,
        mesh=mesh,
        out_type=jax.ShapeDtypeStruct((_E, _H), jnp.float32),
        scratch_types=[
            pltpu.VMEM((_EPW,), jnp.int32),
            pltpu.VMEM((_GCH, _H), jnp.float32),
            pltpu.SemaphoreType.DMA,
        ],
    )
    def _k(x_hbm, idx_hbm, out_hbm, idx_v, rows_v, sem):
        wid = lax.axis_index("s") * _SC_CORES + lax.axis_index("c")
        base = wid * _EPW
        pltpu.sync_copy(idx_hbm.at[pl.ds(base, _EPW)], idx_v)
        for ch in range(_NCH):
            pltpu.async_copy(
                x_hbm.at[idx_v.at[pl.ds(ch * _GCH, _GCH)]], rows_v,
                sem).wait()
            pltpu.sync_copy(rows_v, out_hbm.at[pl.ds(base + ch * _GCH, _GCH)])

    return _k(x, idx_flat)


def _msg_body(a_ref, x_ref, g_ref, wb_ref, mx_ref, s_ref, q_ref):
    r = pl.program_id(0)

    @pl.when(r == 0)
    def _init():
        s_ref[...] = jnp.zeros((1, _H), jnp.float32)
        q_ref[...] = jnp.zeros((1, _H), jnp.float32)

    a = a_ref[...]
    x = x_ref[...]
    wb = wb_ref[...]
    rid = r * _RT + lax.broadcasted_iota(jnp.int32, (_RT, 1), 0)
    valid = rid < _N
    mx = None
    s = jnp.zeros((1, _H), jnp.float32)
    q = jnp.zeros((1, _H), jnp.float32)
    for k in range(_K):
        m = _elu(a + _dotbf(g_ref[:, k, :] - x, wb))
        mx = m if mx is None else jnp.maximum(mx, m)
        mv = jnp.where(valid, m, 0.0)
        s = s + jnp.sum(mv, axis=0, keepdims=True)
        q = q + jnp.sum(mv * mv, axis=0, keepdims=True)
    mx_ref[...] = mx
    s_ref[...] = s_ref[...] + s
    q_ref[...] = q_ref[...] + q


def _msg_call(a, x, g, wb):
    return pl.pallas_call(
        _msg_body,
        grid=(_R,),
        in_specs=[
            pl.BlockSpec((_RT, _H), lambda r: (r, 0)),
            pl.BlockSpec((_RT, _H), lambda r: (r, 0)),
            pl.BlockSpec((_RT, _K, _H), lambda r: (r, 0, 0)),
            pl.BlockSpec((_H, _H), lambda r: (0, 0)),
        ],
        out_specs=[
            pl.BlockSpec((_RT, _H), lambda r: (r, 0)),
            pl.BlockSpec((1, _H), lambda r: (0, 0)),
            pl.BlockSpec((1, _H), lambda r: (0, 0)),
        ],
        out_shape=[
            jax.ShapeDtypeStruct((_NP, _H), jnp.float32),
            jax.ShapeDtypeStruct((1, _H), jnp.float32),
            jax.ShapeDtypeStruct((1, _H), jnp.float32),
        ],
    )(a, x, g, wb)


def _bn(mx_ref, s_ref, q_ref, g_ref, be_ref):
    mu = s_ref[...] / _NKROWS
    var = q_ref[...] / _NKROWS - mu * mu
    return (mx_ref[...] - mu) * lax.rsqrt(var + 1e-5) * g_ref[...] + be_ref[...]


def _fin1_body(mx_ref, s_ref, q_ref, g_ref, be_ref, wt_ref, bc_ref,
               f_ref, a_ref, sq_ref):
    f = _bn(mx_ref, s_ref, q_ref, g_ref, be_ref)
    f_ref[...] = f
    a_ref[...] = _dotbf(f, wt_ref[...]) + bc_ref[...]
    sq_ref[...] = jnp.sum(f * f, axis=1, keepdims=True)


def _fin2_body(mx_ref, s_ref, q_ref, g_ref, be_ref, res_ref, wt_ref,
               bc_ref, f_ref, a_ref, sq_ref):
    f = _bn(mx_ref, s_ref, q_ref, g_ref, be_ref) + res_ref[...]
    f_ref[...] = f
    a_ref[...] = _dotbf(f, wt_ref[...]) + bc_ref[...]
    sq_ref[...] = jnp.sum(f * f, axis=1, keepdims=True)


def _fin3_body(mx_ref, s_ref, q_ref, g_ref, be_ref, res_ref, wo1_ref,
               bo1_ref, wo2_ref, bo2_ref, wo3_ref, bo3_ref, out_ref):
    f = _bn(mx_ref, s_ref, q_ref, g_ref, be_ref) + res_ref[...]
    o = _elu(_dotbf(f, wo1_ref[...]) + bo1_ref[...])
    o = _elu(_dotbf(o, wo2_ref[...]) + bo2_ref[...])
    out_ref[...] = _dotbf(o, wo3_ref[...]) + bo3_ref[...]


_VEC = pl.BlockSpec((1, _H), lambda r: (0, 0))
_ROW = pl.BlockSpec((_RT, _H), lambda r: (r, 0))
_COL1 = pl.BlockSpec((_RT, 1), lambda r: (r, 0))


def _fin_mid_call(mx, s, q, g, be, res, wt, bc):
    sdt = jax.ShapeDtypeStruct((_NP, _H), jnp.float32)
    wspec = pl.BlockSpec((_H, _H), lambda r: (0, 0))
    if res is None:
        body, extra_in, extra_spec = _fin1_body, (), ()
    else:
        body, extra_in, extra_spec = _fin2_body, (res,), (_ROW,)
    return pl.pallas_call(
        body,
        grid=(_R,),
        in_specs=[_ROW, _VEC, _VEC, _VEC, _VEC, *extra_spec, wspec, _VEC],
        out_specs=[_ROW, _ROW, _COL1],
        out_shape=[sdt, sdt, jax.ShapeDtypeStruct((_NP, 1), jnp.float32)],
    )(mx, s, q, g, be, *extra_in, wt, bc)


def _fin_out_call(mx, s, q, g, be, res, wo1, bo1, wo2, bo2, wo3, bo3):
    return pl.pallas_call(
        _fin3_body,
        grid=(_R,),
        in_specs=[
            _ROW, _VEC, _VEC, _VEC, _VEC, _ROW,
            pl.BlockSpec((_H, 32), lambda r: (0, 0)),
            pl.BlockSpec((1, 32), lambda r: (0, 0)),
            pl.BlockSpec((32, 16), lambda r: (0, 0)),
            pl.BlockSpec((1, 16), lambda r: (0, 0)),
            pl.BlockSpec((16, 8), lambda r: (0, 0)),
            pl.BlockSpec((1, 8), lambda r: (0, 0)),
        ],
        out_specs=pl.BlockSpec((_RT, 8), lambda r: (r, 0)),
        out_shape=jax.ShapeDtypeStruct((_NP, 8), jnp.float32),
    )(mx, s, q, g, be, res, wo1, bo1, wo2, bo2, wo3, bo3)


def kernel(x_lc, batch_lc, W_e1, b_e1, W_e2, b_e2, W_c1, b_c1, g_c1, be_c1,
           W_c2, b_c2, g_c2, be_c2, W_c3, b_c3, g_c3, be_c3,
           W_o1, b_o1, W_o2, b_o2, W_o3, b_o3):
    f32 = jnp.float32
    xp = jnp.zeros((_NP, 16), f32).at[:_N, :15].set(x_lc)
    we1 = jnp.zeros((16, _H), f32).at[:15].set(W_e1)
    bp = jnp.full((_NP,), 16, jnp.int32).at[:_N].set(batch_lc.astype(jnp.int32))
    seg = jnp.searchsorted(bp, jnp.arange(18, dtype=jnp.int32)).astype(jnp.int32)
    rs3 = seg[bp].reshape(_R, _RT, 1)
    re3 = seg[bp + 1].reshape(_R, _RT, 1)
    tb = jnp.arange(_R, dtype=jnp.int32) * _RT
    lo_t = seg[bp[tb]] // _CT
    hi_t = (seg[bp[tb + _RT - 1] + 1] - 1) // _CT
    bounds = jnp.stack([lo_t, hi_t]).astype(jnp.int32)

    def row(v):
        return v.reshape(1, -1)

    def layer(x_cur, a, sq, wb, gam, bet, res, nxt):
        sqa = sq.reshape(_R, _RT, 1)
        sqb = sq.reshape(_NJ, 1, _CT)
        idx = _knn_call(bounds, x_cur, sqa, sqb, rs3, re3)
        g = _gather_call(x_cur, idx.reshape(-1)).reshape(_NP, _K, _H)
        mx, s, q = _msg_call(a, x_cur, g, wb)
        return _fin_mid_call(mx, s, q, row(gam), row(bet), res, *nxt) \
            if nxt is not None else (mx, s, q)

    x, a, sq = _enc_call(xp, we1, row(b_e1), W_e2, row(b_e2),
                         W_c1[:_H], row(b_c1))
    f1, a, sq = layer(x, a, sq, W_c1[_H:], g_c1, be_c1, None,
                      (W_c2[:_H], row(b_c2)))
    f2, a, sq = layer(f1, a, sq, W_c2[_H:], g_c2, be_c2, f1,
                      (W_c3[:_H], row(b_c3)))
    mx, s, q = layer(f2, a, sq, W_c3[_H:], None, None, None, None)
    out = _fin_out_call(mx, s, q, row(g_c3), row(be_c3), f2,
                        W_o1, row(b_o1), W_o2, row(b_o2), W_o3, row(b_o3))

    return (out[:_N], batch_lc)
